# Initial kernel scaffold; baseline (speedup 1.0000x reference)
import jax
import jax.numpy as jnp
from jax import lax
from jax.experimental import pallas as pl
from jax.experimental.pallas import tpu as pltpu
from jax.experimental.pallas import tpu_sc as plsc

N = 320000
D = 128
S = 10000
NC = 2
NS = 16
NW = NC * NS
RW = N // NW
C = 128
FULL = RW // C
TAIL = RW - FULL * C
SPT = S // NS
SEG_C = 125
SEG_CH = SPT // SEG_C
CW = 16

_mesh = plsc.VectorSubcoreMesh(core_axis_name="c", subcore_axis_name="s")


def _pass1_body(x_hbm, ids_hbm, psum_hbm, pcnt_hbm,
                acc_sh, cnt_sh, xbuf, ones_b, zrow, zcnt, idsb, idst):
    cid = lax.axis_index("c")
    sid = lax.axis_index("s")
    wid = cid * NS + sid
    base = wid * RW
    seg0 = sid * SPT

    zero16 = jnp.zeros((16,), jnp.float32)
    one16 = jnp.full((16,), 1.0, jnp.float32)

    def fz(r, carry):
        for j in range(D // 16):
            zrow[r, pl.ds(j * 16, 16)] = zero16
        return carry
    lax.fori_loop(0, SEG_C, fz, 0)

    def fzc(r, carry):
        zcnt[r, :] = zero16
        return carry
    lax.fori_loop(0, SPT, fzc, 0)

    def fo(r, carry):
        ones_b[r, :] = one16
        return carry
    lax.fori_loop(0, C, fo, 0)

    for z in range(SEG_CH):
        pltpu.sync_copy(zrow, acc_sh.at[pl.ds(seg0 + z * SEG_C, SEG_C)])
    pltpu.sync_copy(zcnt, cnt_sh.at[pl.ds(seg0, SPT)])
    plsc.subcore_barrier()

    def chunk(g, carry):
        r0 = base + g * C
        pltpu.sync_copy(x_hbm.at[pl.ds(r0, C)], xbuf)
        pltpu.sync_copy(ids_hbm.at[pl.ds(r0, C)], idsb.at[0])
        pltpu.sync_copy(xbuf, acc_sh.at[idsb.at[0]], add=True)
        pltpu.sync_copy(ones_b, cnt_sh.at[idsb.at[0]], add=True)
        return carry
    lax.fori_loop(0, FULL, chunk, 0)

    r0 = base + FULL * C
    pltpu.sync_copy(x_hbm.at[pl.ds(r0, TAIL)], xbuf.at[pl.ds(0, TAIL)])
    pltpu.sync_copy(ids_hbm.at[pl.ds(r0, TAIL)], idst.at[0])
    pltpu.sync_copy(xbuf.at[pl.ds(0, TAIL)], acc_sh.at[idst.at[0]], add=True)
    pltpu.sync_copy(ones_b.at[pl.ds(0, TAIL)], cnt_sh.at[idst.at[0]], add=True)
    plsc.subcore_barrier()

    for z in range(SEG_CH):
        s0 = seg0 + z * SEG_C
        pltpu.sync_copy(acc_sh.at[pl.ds(s0, SEG_C)],
                        psum_hbm.at[pl.ds(cid * S + s0, SEG_C)])
    pltpu.sync_copy(cnt_sh.at[pl.ds(seg0, SPT)],
                    pcnt_hbm.at[pl.ds(cid * S + seg0, SPT)])


_pass1 = pl.kernel(
    _pass1_body,
    out_type=(jax.ShapeDtypeStruct((NC * S, D), jnp.float32),
              jax.ShapeDtypeStruct((NC * S, CW), jnp.float32)),
    mesh=_mesh,
    scratch_types=[
        pltpu.VMEM_SHARED((S, D), jnp.float32),
        pltpu.VMEM_SHARED((S, CW), jnp.float32),
        pltpu.VMEM((C, D), jnp.float32),
        pltpu.VMEM((C, CW), jnp.float32),
        pltpu.VMEM((SEG_C, D), jnp.float32),
        pltpu.VMEM((SPT, CW), jnp.float32),
        pltpu.VMEM((1, C), jnp.int32),
        pltpu.VMEM((1, TAIL), jnp.int32),
    ],
)


def _pass2_body(x_hbm, ids_hbm, psum_hbm, pcnt_hbm, out_hbm,
                mean_sh, p0b, p1b, c0b, c1b, mb, xbuf, mrows, idsb, idst):
    cid = lax.axis_index("c")
    sid = lax.axis_index("s")
    wid = cid * NS + sid
    base = wid * RW
    seg0 = sid * SPT

    one16 = jnp.full((16,), 1.0, jnp.float32)

    for z in range(SEG_CH):
        s0 = seg0 + z * SEG_C
        sl = pl.ds(s0, SEG_C)
        pltpu.sync_copy(psum_hbm.at[pl.ds(s0, SEG_C)], p0b)
        pltpu.sync_copy(psum_hbm.at[pl.ds(S + s0, SEG_C)], p1b)
        pltpu.sync_copy(pcnt_hbm.at[pl.ds(s0, SEG_C)], c0b)
        pltpu.sync_copy(pcnt_hbm.at[pl.ds(S + s0, SEG_C)], c1b)

        def mrow(r, carry):
            cnt = c0b[r, :] + c1b[r, :]
            inv = one16 / jnp.maximum(cnt, one16)
            for j in range(D // 16):
                ds = pl.ds(j * 16, 16)
                mb[r, ds] = (p0b[r, ds] + p1b[r, ds]) * inv
            return carry
        lax.fori_loop(0, SEG_C, mrow, 0)
        pltpu.sync_copy(mb, mean_sh.at[sl])
    plsc.subcore_barrier()

    def chunk(g, carry):
        r0 = base + g * C
        pltpu.sync_copy(x_hbm.at[pl.ds(r0, C)], xbuf)
        pltpu.sync_copy(ids_hbm.at[pl.ds(r0, C)], idsb.at[0])
        pltpu.sync_copy(mean_sh.at[idsb.at[0]], mrows)

        def sub(r, carry2):
            for j in range(D // 16):
                ds = pl.ds(j * 16, 16)
                xbuf[r, ds] = xbuf[r, ds] - mrows[r, ds]
            return carry2
        lax.fori_loop(0, C, sub, 0)
        pltpu.sync_copy(xbuf, out_hbm.at[pl.ds(r0, C)])
        return carry
    lax.fori_loop(0, FULL, chunk, 0)

    r0 = base + FULL * C
    pltpu.sync_copy(x_hbm.at[pl.ds(r0, TAIL)], xbuf.at[pl.ds(0, TAIL)])
    pltpu.sync_copy(ids_hbm.at[pl.ds(r0, TAIL)], idst.at[0])
    pltpu.sync_copy(mean_sh.at[idst.at[0]], mrows.at[pl.ds(0, TAIL)])

    def subt(r, carry):
        for j in range(D // 16):
            ds = pl.ds(j * 16, 16)
            xbuf[r, ds] = xbuf[r, ds] - mrows[r, ds]
        return carry
    lax.fori_loop(0, TAIL, subt, 0)
    pltpu.sync_copy(xbuf.at[pl.ds(0, TAIL)], out_hbm.at[pl.ds(r0, TAIL)])


_pass2 = pl.kernel(
    _pass2_body,
    out_type=jax.ShapeDtypeStruct((N, D), jnp.float32),
    mesh=_mesh,
    scratch_types=[
        pltpu.VMEM_SHARED((S, D), jnp.float32),
        pltpu.VMEM((SEG_C, D), jnp.float32),
        pltpu.VMEM((SEG_C, D), jnp.float32),
        pltpu.VMEM((SEG_C, CW), jnp.float32),
        pltpu.VMEM((SEG_C, CW), jnp.float32),
        pltpu.VMEM((SEG_C, D), jnp.float32),
        pltpu.VMEM((C, D), jnp.float32),
        pltpu.VMEM((C, D), jnp.float32),
        pltpu.VMEM((1, C), jnp.int32),
        pltpu.VMEM((1, TAIL), jnp.int32),
    ],
)


def kernel(x_pos, segment_ids):
    ids = segment_ids.astype(jnp.int32)
    psum, pcnt = _pass1(x_pos, ids)
    return _pass2(x_pos, ids, psum, pcnt)


# speedup vs baseline: 1.5975x; 1.5975x over previous
import jax
import jax.numpy as jnp
from jax import lax
from jax.experimental import pallas as pl
from jax.experimental.pallas import tpu as pltpu
from jax.experimental.pallas import tpu_sc as plsc

N = 320000
D = 128
S = 10000
NC = 2
NS = 16
NW = NC * NS
SEG_W = 320
T = SEG_W + 1
DUMP = SEG_W
C = 64
IDSB = 80
NB = N // 16

_mesh = plsc.VectorSubcoreMesh(core_axis_name="c", subcore_axis_name="s")


def _body(x_hbm, ids_hbm, out_hbm, acc, cnt, xbuf, idsbuf, pbuf):
    cid = lax.axis_index("c")
    sid = lax.axis_index("s")
    w = cid * NS + sid

    zero16 = jnp.zeros((16,), jnp.float32)
    one16 = jnp.full((16,), 1.0, jnp.float32)

    def zrow(t, carry):
        for j in range(D // 16):
            acc[t, pl.ds(j * 16, 16)] = zero16
        cnt[t, :] = zero16
        return carry
    lax.fori_loop(0, T, zrow, 0)

    def lower_bound(target):
        def bs(it, lohi):
            lo, hi = lohi
            mid = (lo + hi) // 2
            pltpu.sync_copy(ids_hbm.at[pl.ds(mid * 16, 16)], pbuf)
            bmax = pbuf[pl.ds(0, 16)][15]
            below = bmax < target
            return (jnp.where(below, mid + 1, lo), jnp.where(below, hi, mid))
        b, _ = lax.fori_loop(0, 15, bs, (0, NB))
        bc = jnp.minimum(b, NB - 1)
        pltpu.sync_copy(ids_hbm.at[pl.ds(bc * 16, 16)], pbuf)
        v = pbuf[pl.ds(0, 16)]
        f = jnp.int32(16)
        for l in reversed(range(16)):
            f = jnp.where(v[l] >= target, jnp.int32(l), f)
        return jnp.where(b >= NB, N, bc * 16 + f)

    lo_row = lower_bound(w * SEG_W)
    hi_row = lower_bound((w + 1) * SEG_W)
    nrows = hi_row - lo_row
    nfull = nrows // C
    ntail = nrows - nfull * C

    def load_chunk(cstart):
        pltpu.sync_copy(x_hbm.at[pl.ds(cstart * D, C * D)], xbuf)
        a8 = jnp.minimum(pl.multiple_of((cstart // 8) * 8, 8), N - IDSB)
        pltpu.sync_copy(ids_hbm.at[pl.ds(a8, IDSB)], idsbuf)
        return cstart - a8

    def accum_chunk(cstart, skew, vlo, vhi):
        def grp(k, carry):
            vec = idsbuf[pl.ds(skew + k * 16, 16)]
            for l in range(16):
                i = k * 16 + l
                seg = vec[l]
                ridx = cstart + i
                valid = (ridx >= vlo) & (ridx < vhi)
                t = jnp.where(valid, seg - w * SEG_W, DUMP)
                for j in range(D // 16):
                    sl = pl.ds(j * 16, 16)
                    acc[t, sl] = acc[t, sl] + xbuf[pl.ds(i * D + j * 16, 16)]
                cnt[t, :] = cnt[t, :] + one16
            return carry
        lax.fori_loop(0, C // 16, grp, 0)

    def out_chunk(cstart, skew):
        def grp(k, carry):
            vec = idsbuf[pl.ds(skew + k * 16, 16)]
            for l in range(16):
                i = k * 16 + l
                t = jnp.clip(vec[l] - w * SEG_W, 0, DUMP)
                for j in range(D // 16):
                    sl = pl.ds(j * 16, 16)
                    xbuf[pl.ds(i * D + j * 16, 16)] = (
                        xbuf[pl.ds(i * D + j * 16, 16)] - acc[t, sl])
            return carry
        lax.fori_loop(0, C // 16, grp, 0)

    def acc_full(g, carry):
        row0 = lo_row + g * C
        skew = load_chunk(row0)
        accum_chunk(row0, skew, row0, row0 + C)
        return carry
    lax.fori_loop(0, nfull, acc_full, 0)

    @pl.when(ntail > 0)
    def _():
        row0 = lo_row + nfull * C
        cstart = jnp.minimum(row0, N - C)
        skew = load_chunk(cstart)
        accum_chunk(cstart, skew, row0, hi_row)

    def mean_row(t, carry):
        inv = one16 / jnp.maximum(cnt[t, :], one16)
        for j in range(D // 16):
            sl = pl.ds(j * 16, 16)
            acc[t, sl] = acc[t, sl] * inv
        return carry
    lax.fori_loop(0, SEG_W, mean_row, 0)

    def out_full(g, carry):
        row0 = lo_row + g * C
        skew = load_chunk(row0)
        out_chunk(row0, skew)
        pltpu.sync_copy(xbuf, out_hbm.at[pl.ds(row0 * D, C * D)])
        return carry
    lax.fori_loop(0, nfull, out_full, 0)

    @pl.when(ntail > 0)
    def _():
        row0 = lo_row + nfull * C
        cstart = jnp.minimum(row0, N - C)
        skew = load_chunk(cstart)
        out_chunk(cstart, skew)
        shift = row0 - cstart

        def wrow(r, carry):
            pltpu.sync_copy(xbuf.at[pl.ds((shift + r) * D, D)],
                            out_hbm.at[pl.ds((row0 + r) * D, D)])
            return carry
        lax.fori_loop(0, ntail, wrow, 0)


_sc_kernel = pl.kernel(
    _body,
    out_type=jax.ShapeDtypeStruct((N * D,), jnp.float32),
    mesh=_mesh,
    scratch_types=[
        pltpu.VMEM((T, D), jnp.float32),
        pltpu.VMEM((T, 16), jnp.float32),
        pltpu.VMEM((C * D,), jnp.float32),
        pltpu.VMEM((IDSB,), jnp.int32),
        pltpu.VMEM((16,), jnp.int32),
    ],
)


def kernel(x_pos, segment_ids):
    ids = segment_ids.astype(jnp.int32)
    out_flat = _sc_kernel(x_pos.reshape(N * D), ids)
    return out_flat.reshape(N, D)


# speedup vs baseline: 1.9996x; 1.2517x over previous
import jax
import jax.numpy as jnp
from jax import lax
from jax.experimental import pallas as pl
from jax.experimental.pallas import tpu as pltpu
from jax.experimental.pallas import tpu_sc as plsc

N = 320000
D = 128
S = 10000
NC = 2
NS = 16
NW = NC * NS
SEG_W = 320
T = SEG_W + 1
DUMP = SEG_W
C = 64
IDSB = 80
NB = N // 16

_mesh = plsc.VectorSubcoreMesh(core_axis_name="c", subcore_axis_name="s")


def _body(x_hbm, ids_hbm, out_hbm, acc, cnt, xbuf, idsbuf, pbuf):
    cid = lax.axis_index("c")
    sid = lax.axis_index("s")
    w = cid * NS + sid

    zero16 = jnp.zeros((16,), jnp.float32)
    one16 = jnp.full((16,), 1.0, jnp.float32)

    def zrow(t, carry):
        for j in range(D // 16):
            acc[t, pl.ds(j * 16, 16)] = zero16
        cnt[t, :] = zero16
        return carry
    lax.fori_loop(0, T, zrow, 0)

    def lower_bound(target):
        def bs(it, lohi):
            lo, hi = lohi
            mid = (lo + hi) // 2
            pltpu.sync_copy(ids_hbm.at[pl.ds(mid * 16, 16)], pbuf)
            bmax = pbuf[pl.ds(0, 16)][15]
            below = bmax < target
            return (jnp.where(below, mid + 1, lo), jnp.where(below, hi, mid))
        b, _ = lax.fori_loop(0, 15, bs, (0, NB))
        bc = jnp.minimum(b, NB - 1)
        pltpu.sync_copy(ids_hbm.at[pl.ds(bc * 16, 16)], pbuf)
        v = pbuf[pl.ds(0, 16)]
        f = jnp.int32(16)
        for l in reversed(range(16)):
            f = jnp.where(v[l] >= target, jnp.int32(l), f)
        return jnp.where(b >= NB, N, bc * 16 + f)

    lo_row = lower_bound(w * SEG_W)
    hi_row = lower_bound((w + 1) * SEG_W)
    nrows = hi_row - lo_row
    nfull = nrows // C
    ntail = nrows - nfull * C

    def load_chunk(cstart):
        pltpu.sync_copy(x_hbm.at[pl.ds(cstart * D, C * D)], xbuf)
        a8 = jnp.minimum(pl.multiple_of((cstart // 8) * 8, 8), N - IDSB)
        pltpu.sync_copy(ids_hbm.at[pl.ds(a8, IDSB)], idsbuf)
        return cstart - a8

    NJ = D // 16

    def flush(pt, svec, cvec):
        for j in range(NJ):
            sl = pl.ds(j * 16, 16)
            acc[pt, sl] = acc[pt, sl] + svec[j]
        cnt[pt, :] = cnt[pt, :] + cvec

    def accum_chunk(cstart, skew, vlo, vhi, carry):
        def grp(k, car):
            prev_t = car[0]
            c16 = car[1]
            s = list(car[2:])
            vec = idsbuf[pl.ds(skew + k * 16, 16)]
            for l in range(16):
                i = k * 16 + l
                seg = vec[l]
                ridx = cstart + i
                valid = (ridx >= vlo) & (ridx < vhi)
                t = jnp.where(valid, seg - w * SEG_W, DUMP)
                change = t != prev_t
                sl_ = list(s)
                pt_ = prev_t
                cc_ = c16

                @pl.when(change)
                def _():
                    flush(pt_, sl_, cc_)
                keep = jnp.where(change, jnp.float32(0), jnp.float32(1))
                keep16 = jnp.broadcast_to(keep, (16,))
                xrow = [xbuf[pl.ds(i * D + j * 16, 16)] for j in range(NJ)]
                s = [s[j] * keep16 + xrow[j] for j in range(NJ)]
                c16 = c16 * keep16 + one16
                prev_t = t
            return (prev_t, c16, *s)
        return lax.fori_loop(0, C // 16, grp, carry)

    def out_chunk(cstart, skew, carry):
        def grp(k, car):
            vec = idsbuf[pl.ds(skew + k * 16, 16)]
            for l in range(16):
                i = k * 16 + l
                t = jnp.clip(vec[l] - w * SEG_W, 0, DUMP)
                for j in range(NJ):
                    sl = pl.ds(i * D + j * 16, 16)
                    xbuf[sl] = xbuf[sl] - acc[t, pl.ds(j * 16, 16)]
            return car
        return lax.fori_loop(0, C // 16, grp, carry)

    carry = (jnp.int32(DUMP), zero16) + tuple([zero16] * NJ)

    def acc_full(g, car):
        row0 = lo_row + g * C
        skew = load_chunk(row0)
        return accum_chunk(row0, skew, row0, row0 + C, car)
    carry = lax.fori_loop(0, nfull, acc_full, carry)

    trow0 = lo_row + nfull * C
    tstart = jnp.minimum(trow0, N - C)
    tskew = load_chunk(tstart)
    carry = accum_chunk(tstart, tskew, trow0, hi_row, carry)
    flush(carry[0], list(carry[2:]), carry[1])

    def mean_row(t, carry):
        inv = one16 / jnp.maximum(cnt[t, :], one16)
        for j in range(D // 16):
            sl = pl.ds(j * 16, 16)
            acc[t, sl] = acc[t, sl] * inv
        return carry
    lax.fori_loop(0, SEG_W, mean_row, 0)

    ocarry = 0

    def out_full(g, car):
        row0 = lo_row + g * C
        skew = load_chunk(row0)
        car = out_chunk(row0, skew, car)
        pltpu.sync_copy(xbuf, out_hbm.at[pl.ds(row0 * D, C * D)])
        return car
    ocarry = lax.fori_loop(0, nfull, out_full, ocarry)

    orow0 = lo_row + nfull * C
    ostart = jnp.minimum(orow0, N - C)
    oskew = load_chunk(ostart)
    out_chunk(ostart, oskew, ocarry)
    oshift = orow0 - ostart

    def wrow(r, car):
        pltpu.sync_copy(xbuf.at[pl.ds((oshift + r) * D, D)],
                        out_hbm.at[pl.ds((orow0 + r) * D, D)])
        return car
    lax.fori_loop(0, ntail, wrow, 0)


_sc_kernel = pl.kernel(
    _body,
    out_type=jax.ShapeDtypeStruct((N * D,), jnp.float32),
    mesh=_mesh,
    scratch_types=[
        pltpu.VMEM((T, D), jnp.float32),
        pltpu.VMEM((T, 16), jnp.float32),
        pltpu.VMEM((C * D,), jnp.float32),
        pltpu.VMEM((IDSB,), jnp.int32),
        pltpu.VMEM((16,), jnp.int32),
    ],
)


def kernel(x_pos, segment_ids):
    ids = segment_ids.astype(jnp.int32)
    out_flat = _sc_kernel(x_pos.reshape(N * D), ids)
    return out_flat.reshape(N, D)


# speedup vs baseline: 2.3660x; 1.1832x over previous
import jax
import jax.numpy as jnp
from jax import lax
from jax.experimental import pallas as pl
from jax.experimental.pallas import tpu as pltpu
from jax.experimental.pallas import tpu_sc as plsc

N = 320000
D = 128
S = 10000
NC = 2
NS = 16
NW = NC * NS
SEG_W = 320
T = SEG_W + 1
DUMP = SEG_W
C = 64
IDSB = 80
NB = N // 16

_mesh = plsc.VectorSubcoreMesh(core_axis_name="c", subcore_axis_name="s")


def _body(x_hbm, ids_hbm, out_hbm, acc, cnt, xbuf, idsbuf, pbuf):
    cid = lax.axis_index("c")
    sid = lax.axis_index("s")
    w = cid * NS + sid

    zero16 = jnp.zeros((16,), jnp.float32)
    one16 = jnp.full((16,), 1.0, jnp.float32)

    def zrow(t, carry):
        for j in range(D // 16):
            acc[t, pl.ds(j * 16, 16)] = zero16
        cnt[t, :] = zero16
        return carry
    lax.fori_loop(0, T, zrow, 0)

    def lower_bound(target):
        def bs(it, lohi):
            lo, hi = lohi
            mid = (lo + hi) // 2
            pltpu.sync_copy(ids_hbm.at[pl.ds(mid * 16, 16)], pbuf)
            bmax = pbuf[pl.ds(0, 16)][15]
            below = bmax < target
            return (jnp.where(below, mid + 1, lo), jnp.where(below, hi, mid))
        b, _ = lax.fori_loop(0, 15, bs, (0, NB))
        bc = jnp.minimum(b, NB - 1)
        pltpu.sync_copy(ids_hbm.at[pl.ds(bc * 16, 16)], pbuf)
        v = pbuf[pl.ds(0, 16)]
        f = jnp.int32(16)
        for l in reversed(range(16)):
            f = jnp.where(v[l] >= target, jnp.int32(l), f)
        return jnp.where(b >= NB, N, bc * 16 + f)

    lo_row = lower_bound(w * SEG_W)
    hi_row = lower_bound((w + 1) * SEG_W)
    nrows = hi_row - lo_row
    nfull = nrows // C
    ntail = nrows - nfull * C

    def load_chunk(cstart):
        pltpu.sync_copy(x_hbm.at[pl.ds(cstart * D, C * D)], xbuf)
        a8 = jnp.minimum(pl.multiple_of((cstart // 8) * 8, 8), N - IDSB)
        pltpu.sync_copy(ids_hbm.at[pl.ds(a8, IDSB)], idsbuf)
        return cstart - a8

    NJ = D // 16

    def flush(pt, svec, cvec):
        for j in range(NJ):
            sl = pl.ds(j * 16, 16)
            acc[pt, sl] = acc[pt, sl] + svec[j]
        cnt[pt, :] = cnt[pt, :] + cvec

    def accum_chunk(cstart, skew, vlo, vhi, carry):
        def grp(k, car):
            prev_t = car[0]
            c16 = car[1]
            s = list(car[2:])
            vec = idsbuf[pl.ds(skew + k * 16, 16)]
            for l in range(16):
                i = k * 16 + l
                seg = vec[l]
                ridx = cstart + i
                valid = (ridx >= vlo) & (ridx < vhi)
                t = jnp.where(valid, seg - w * SEG_W, DUMP)
                change = t != prev_t
                sl_ = list(s)
                pt_ = prev_t
                cc_ = c16

                @pl.when(change)
                def _():
                    flush(pt_, sl_, cc_)
                keep = jnp.where(change, jnp.float32(0), jnp.float32(1))
                keep16 = jnp.broadcast_to(keep, (16,))
                xrow = [xbuf[pl.ds(i * D + j * 16, 16)] for j in range(NJ)]
                s = [s[j] * keep16 + xrow[j] for j in range(NJ)]
                c16 = c16 * keep16 + one16
                prev_t = t
            return (prev_t, c16, *s)
        return lax.fori_loop(0, C // 16, grp, carry)

    def out_chunk(cstart, skew, carry):
        def grp(k, car):
            vec = idsbuf[pl.ds(skew + k * 16, 16)]
            same = vec[0] == vec[15]

            @pl.when(same)
            def _():
                t = jnp.clip(vec[0] - w * SEG_W, 0, DUMP)
                m = [acc[t, pl.ds(j * 16, 16)] for j in range(NJ)]
                for l in range(16):
                    i = k * 16 + l
                    for j in range(NJ):
                        sl = pl.ds(i * D + j * 16, 16)
                        xbuf[sl] = xbuf[sl] - m[j]

            @pl.when(jnp.logical_not(same))
            def _():
                for l in range(16):
                    i = k * 16 + l
                    t = jnp.clip(vec[l] - w * SEG_W, 0, DUMP)
                    for j in range(NJ):
                        sl = pl.ds(i * D + j * 16, 16)
                        xbuf[sl] = xbuf[sl] - acc[t, pl.ds(j * 16, 16)]
            return car
        return lax.fori_loop(0, C // 16, grp, carry)

    carry = (jnp.int32(DUMP), zero16) + tuple([zero16] * NJ)

    def acc_full(g, car):
        row0 = lo_row + g * C
        skew = load_chunk(row0)
        return accum_chunk(row0, skew, row0, row0 + C, car)
    carry = lax.fori_loop(0, nfull, acc_full, carry)

    trow0 = lo_row + nfull * C
    tstart = jnp.minimum(trow0, N - C)
    tskew = load_chunk(tstart)
    carry = accum_chunk(tstart, tskew, trow0, hi_row, carry)
    flush(carry[0], list(carry[2:]), carry[1])

    def mean_row(t, carry):
        inv = one16 / jnp.maximum(cnt[t, :], one16)
        for j in range(D // 16):
            sl = pl.ds(j * 16, 16)
            acc[t, sl] = acc[t, sl] * inv
        return carry
    lax.fori_loop(0, SEG_W, mean_row, 0)

    ocarry = 0

    def out_full(g, car):
        row0 = lo_row + g * C
        skew = load_chunk(row0)
        car = out_chunk(row0, skew, car)
        pltpu.sync_copy(xbuf, out_hbm.at[pl.ds(row0 * D, C * D)])
        return car
    ocarry = lax.fori_loop(0, nfull, out_full, ocarry)

    orow0 = lo_row + nfull * C
    ostart = jnp.minimum(orow0, N - C)
    oskew = load_chunk(ostart)
    out_chunk(ostart, oskew, ocarry)
    oshift = orow0 - ostart

    def wrow(r, car):
        pltpu.sync_copy(xbuf.at[pl.ds((oshift + r) * D, D)],
                        out_hbm.at[pl.ds((orow0 + r) * D, D)])
        return car
    lax.fori_loop(0, ntail, wrow, 0)


_sc_kernel = pl.kernel(
    _body,
    out_type=jax.ShapeDtypeStruct((N * D,), jnp.float32),
    mesh=_mesh,
    scratch_types=[
        pltpu.VMEM((T, D), jnp.float32),
        pltpu.VMEM((T, 16), jnp.float32),
        pltpu.VMEM((C * D,), jnp.float32),
        pltpu.VMEM((IDSB,), jnp.int32),
        pltpu.VMEM((16,), jnp.int32),
    ],
)


def kernel(x_pos, segment_ids):
    ids = segment_ids.astype(jnp.int32)
    out_flat = _sc_kernel(x_pos.reshape(N * D), ids)
    return out_flat.reshape(N, D)


# speedup vs baseline: 4.5450x; 1.9210x over previous
import jax
import jax.numpy as jnp
from jax import lax
from jax.experimental import pallas as pl
from jax.experimental.pallas import tpu as pltpu
from jax.experimental.pallas import tpu_sc as plsc

N = 320000
D = 128
S = 10000
NC = 2
NS = 16
NW = NC * NS
SEG_W = 320
T = SEG_W + 1
DUMP = SEG_W
C = 64
IDSB = 80
NB = N // 16

_mesh = plsc.VectorSubcoreMesh(core_axis_name="c", subcore_axis_name="s")


def _body(x_hbm, ids_hbm, out_hbm, acc, cnt, xb0, xb1, ib0, ib1, pbuf,
          sem0, sem1):
    cid = lax.axis_index("c")
    sid = lax.axis_index("s")
    w = cid * NS + sid

    zero16 = jnp.zeros((16,), jnp.float32)
    one16 = jnp.full((16,), 1.0, jnp.float32)

    def zrow(t, carry):
        for j in range(D // 16):
            acc[t, pl.ds(j * 16, 16)] = zero16
        cnt[t, :] = zero16
        return carry
    lax.fori_loop(0, T, zrow, 0)

    def lower_bound(target):
        def bs(it, lohi):
            lo, hi = lohi
            mid = (lo + hi) // 2
            pltpu.sync_copy(ids_hbm.at[pl.ds(mid * 16, 16)], pbuf)
            bmax = pbuf[pl.ds(0, 16)][15]
            below = bmax < target
            return (jnp.where(below, mid + 1, lo), jnp.where(below, hi, mid))
        b, _ = lax.fori_loop(0, 15, bs, (0, NB))
        bc = jnp.minimum(b, NB - 1)
        pltpu.sync_copy(ids_hbm.at[pl.ds(bc * 16, 16)], pbuf)
        v = pbuf[pl.ds(0, 16)]
        f = jnp.int32(16)
        for l in reversed(range(16)):
            f = jnp.where(v[l] >= target, jnp.int32(l), f)
        return jnp.where(b >= NB, N, bc * 16 + f)

    lo_row = lower_bound(w * SEG_W)
    hi_row = lower_bound((w + 1) * SEG_W)
    nrows = hi_row - lo_row
    nfull = nrows // C
    ntail = nrows - nfull * C

    def cstart_of(g):
        return jnp.minimum(lo_row + g * C, N - C)

    def aoff_of(cs):
        return jnp.minimum(pl.multiple_of((cs // 8) * 8, 8), N - IDSB)

    def vlo_of(g):
        return jnp.minimum(lo_row + g * C, hi_row)

    def issue_load(g, xb, ib, sem):
        cs = cstart_of(g)
        pltpu.async_copy(x_hbm.at[pl.ds(cs * D, C * D)], xb, sem)
        pltpu.async_copy(ids_hbm.at[pl.ds(aoff_of(cs), IDSB)], ib, sem)

    def wait_load(xb, ib, sem):
        pltpu.make_async_copy(x_hbm.at[pl.ds(0, C * D)], xb, sem).wait()
        pltpu.make_async_copy(ids_hbm.at[pl.ds(0, IDSB)], ib, sem).wait()

    BUFS = ((xb0, ib0, sem0), (xb1, ib1, sem1))

    NJ = D // 16

    def flush(pt, svec, cvec):
        for j in range(NJ):
            sl = pl.ds(j * 16, 16)
            acc[pt, sl] = acc[pt, sl] + svec[j]
        cnt[pt, :] = cnt[pt, :] + cvec

    def accum_chunk(xbuf, idsbuf, cstart, skew, vlo, vhi, carry):
        def grp(k, car):
            prev_t = car[0]
            c16 = car[1]
            s = list(car[2:])
            vec = idsbuf[pl.ds(skew + k * 16, 16)]
            for l in range(16):
                i = k * 16 + l
                seg = vec[l]
                ridx = cstart + i
                valid = (ridx >= vlo) & (ridx < vhi)
                t = jnp.where(valid, seg - w * SEG_W, DUMP)
                change = t != prev_t
                sl_ = list(s)
                pt_ = prev_t
                cc_ = c16

                @pl.when(change)
                def _():
                    flush(pt_, sl_, cc_)
                keep = jnp.where(change, jnp.float32(0), jnp.float32(1))
                keep16 = jnp.broadcast_to(keep, (16,))
                xrow = [xbuf[pl.ds(i * D + j * 16, 16)] for j in range(NJ)]
                s = [s[j] * keep16 + xrow[j] for j in range(NJ)]
                c16 = c16 * keep16 + one16
                prev_t = t
            return (prev_t, c16, *s)
        return lax.fori_loop(0, C // 16, grp, carry)

    def out_chunk(xbuf, idsbuf, cstart, skew, carry):
        def grp(k, car):
            vec = idsbuf[pl.ds(skew + k * 16, 16)]
            same = vec[0] == vec[15]

            @pl.when(same)
            def _():
                t = jnp.clip(vec[0] - w * SEG_W, 0, DUMP)
                m = [acc[t, pl.ds(j * 16, 16)] for j in range(NJ)]
                for l in range(16):
                    i = k * 16 + l
                    for j in range(NJ):
                        sl = pl.ds(i * D + j * 16, 16)
                        xbuf[sl] = xbuf[sl] - m[j]

            @pl.when(jnp.logical_not(same))
            def _():
                for l in range(16):
                    i = k * 16 + l
                    t = jnp.clip(vec[l] - w * SEG_W, 0, DUMP)
                    for j in range(NJ):
                        sl = pl.ds(i * D + j * 16, 16)
                        xbuf[sl] = xbuf[sl] - acc[t, pl.ds(j * 16, 16)]
            return car
        return lax.fori_loop(0, C // 16, grp, carry)

    nchunks = nfull + 1
    npairs = (nchunks + 1) // 2

    carry = (jnp.int32(DUMP), zero16) + tuple([zero16] * NJ)
    issue_load(0, *BUFS[0])
    issue_load(1, *BUFS[1])

    def acc_pair(p, car):
        for b in range(2):
            xb, ib, sem = BUFS[b]
            g = 2 * p + b
            wait_load(xb, ib, sem)
            cs = cstart_of(g)
            skew = cs - aoff_of(cs)
            car = accum_chunk(xb, ib, cs, skew, vlo_of(g), vlo_of(g + 1),
                              car)
            issue_load(g + 2, xb, ib, sem)
        return car
    carry = lax.fori_loop(0, npairs, acc_pair, carry)
    flush(carry[0], list(carry[2:]), carry[1])
    wait_load(*BUFS[0])
    wait_load(*BUFS[1])

    def mean_row(t, carry):
        inv = one16 / jnp.maximum(cnt[t, :], one16)
        for j in range(D // 16):
            sl = pl.ds(j * 16, 16)
            acc[t, sl] = acc[t, sl] * inv
        return carry
    lax.fori_loop(0, SEG_W, mean_row, 0)

    issue_load(0, *BUFS[0])
    issue_load(1, *BUFS[1])

    def out_pair(p, car):
        for b in range(2):
            xb, ib, sem = BUFS[b]
            g = 2 * p + b
            wait_load(xb, ib, sem)
            cs = cstart_of(g)
            skew = cs - aoff_of(cs)
            out_chunk(xb, ib, cs, skew, 0)
            vlo = vlo_of(g)
            wlen = vlo_of(g + 1) - vlo

            @pl.when(wlen == C)
            def _():
                pltpu.sync_copy(xb, out_hbm.at[pl.ds(vlo * D, C * D)])

            @pl.when(wlen < C)
            def _():
                shift = vlo - cs

                def wrow(r, car2):
                    pltpu.sync_copy(
                        xb.at[pl.ds((shift + r) * D, D)],
                        out_hbm.at[pl.ds((vlo + r) * D, D)])
                    return car2
                lax.fori_loop(0, wlen, wrow, 0)
            issue_load(g + 2, xb, ib, sem)
        return car
    lax.fori_loop(0, npairs, out_pair, 0)
    wait_load(*BUFS[0])
    wait_load(*BUFS[1])


_sc_kernel = pl.kernel(
    _body,
    out_type=jax.ShapeDtypeStruct((N * D,), jnp.float32),
    mesh=_mesh,
    scratch_types=[
        pltpu.VMEM((T, D), jnp.float32),
        pltpu.VMEM((T, 16), jnp.float32),
        pltpu.VMEM((C * D,), jnp.float32),
        pltpu.VMEM((C * D,), jnp.float32),
        pltpu.VMEM((IDSB,), jnp.int32),
        pltpu.VMEM((IDSB,), jnp.int32),
        pltpu.VMEM((16,), jnp.int32),
        pltpu.SemaphoreType.DMA,
        pltpu.SemaphoreType.DMA,
    ],
)


def kernel(x_pos, segment_ids):
    ids = segment_ids.astype(jnp.int32)
    out_flat = _sc_kernel(x_pos.reshape(N * D), ids)
    return out_flat.reshape(N, D)
